# baseline (speedup 1.0000x reference)
import jax
import jax.numpy as jnp
from jax.experimental import pallas as pl
from jax.experimental.pallas import tpu as pltpu

_K = 10
_GAMMA = 8.0
_N = 10000
_NP = 10240
_D = 128
_T = 256
_TA = 16
_TQ = 2 * _TA
_CW = 512
_NC = _NP // _CW
_NR = _D // 8


def _body(a1_ref, a2_ref, g1p_ref, g2p_ref, loss_ref,
          q1_ref, q2_ref, bq_ref, dist_ref, accum_ref, sg_ref,
          g1t_ref, g2t_ref):
    t = pl.program_id(0)

    @pl.when(t == 0)
    def _init():
        accum_ref[...] = jnp.zeros_like(accum_ref)
        g1t_ref[:, 0:_N] = g1p_ref[...].T
        g2t_ref[:, 0:_N] = g2p_ref[...].T
        padfill = jnp.full((_D, _NP - _N), jnp.inf, dtype=jnp.float32)
        g1t_ref[:, _N:] = padfill
        g2t_ref[:, _N:] = padfill
        sg_ref[0:1, :] = jnp.sum(g2t_ref[...], axis=0, keepdims=True)
        sg_ref[1:2, :] = jnp.sum(g1t_ref[...], axis=0, keepdims=True)

    for i in range(_TA):
        i1 = a1_ref[t * _TA + i]
        i2 = a2_ref[t * _TA + i]
        q1_ref[pl.ds(i, 1), :] = g1p_ref[pl.ds(i1, 1), :]
        q2_ref[pl.ds(i, 1), :] = g2p_ref[pl.ds(i2, 1), :]

    q1 = q1_ref[...]
    q2 = q2_ref[...]
    dvec = jnp.sum(jnp.abs(q1 - q2), axis=1, keepdims=True) + _GAMMA
    q1t = q1.T
    q2t = q2.T
    for j in range(_TA):
        bq_ref[j] = jnp.broadcast_to(q1t[:, j:j + 1], (_D, _CW))
        bq_ref[_TA + j] = jnp.broadcast_to(q2t[:, j:j + 1], (_D, _CW))

    sq1 = jnp.sum(q1, axis=1, keepdims=True)
    sq2 = jnp.sum(q2, axis=1, keepdims=True)
    for d, (gref, sq) in enumerate(((g2t_ref, sq1), (g1t_ref, sq2))):
        for g0 in range(0, _TA, 8):
            def chunk(c, _, gref=gref, sq=sq, d=d, g0=g0):
                lanes = pl.ds(c * _CW, _CW)
                accs = [None] * 8
                for r in range(_NR):
                    blk = gref[pl.ds(8 * r, 8), lanes]
                    for j in range(8):
                        bq = bq_ref[d * _TA + g0 + j, pl.ds(8 * r, 8), :]
                        term = jnp.minimum(blk, bq)
                        accs[j] = term if accs[j] is None else accs[j] + term
                sg = sg_ref[pl.ds(d, 1), lanes]
                for j in range(8):
                    red = jnp.sum(accs[j], axis=0, keepdims=True)
                    dist_ref[pl.ds(d * _TA + g0 + j, 1), lanes] = (
                        sq[g0 + j:g0 + j + 1, :] + sg - 2.0 * red)
                return 0
            jax.lax.fori_loop(0, _NC, chunk, 0, unroll=4)

    dist = dist_ref[...]
    dmat = jnp.concatenate([dvec, dvec], axis=0)
    rem = jnp.full((_TQ, 1), float(_K), dtype=jnp.float32)
    av = jnp.zeros((_TQ, 1), dtype=jnp.float32)
    for _ in range(_K):
        m = jnp.min(dist, axis=1, keepdims=True)
        eq = dist == m
        cnt = jnp.sum(eq.astype(jnp.float32), axis=1, keepdims=True)
        dist = jnp.where(eq, jnp.inf, dist)
        w = jnp.minimum(cnt, rem)
        rem = rem - w
        av = av + jnp.maximum(dmat - m, 0.0) * w
    accum_ref[...] += av

    @pl.when(t == pl.num_programs(0) - 1)
    def _fin():
        loss_ref[0, 0] = jnp.sum(accum_ref[...]) / (_T * _K)


def kernel(out1, out2, anchor1, anchor2):
    out = pl.pallas_call(
        _body,
        grid=(_T // _TA,),
        in_specs=[
            pl.BlockSpec(memory_space=pltpu.SMEM),
            pl.BlockSpec(memory_space=pltpu.SMEM),
            pl.BlockSpec((_N, _D), lambda t: (0, 0)),
            pl.BlockSpec((_N, _D), lambda t: (0, 0)),
        ],
        out_specs=pl.BlockSpec(memory_space=pltpu.SMEM),
        out_shape=jax.ShapeDtypeStruct((1, 1), jnp.float32),
        scratch_shapes=[
            pltpu.VMEM((_TA, _D), jnp.float32),
            pltpu.VMEM((_TA, _D), jnp.float32),
            pltpu.VMEM((_TQ, _D, _CW), jnp.float32),
            pltpu.VMEM((_TQ, _NP), jnp.float32),
            pltpu.VMEM((_TQ, 1), jnp.float32),
            pltpu.VMEM((2, _NP), jnp.float32),
            pltpu.VMEM((_D, _NP), jnp.float32),
            pltpu.VMEM((_D, _NP), jnp.float32),
        ],
    )(anchor1, anchor2, out1, out2)
    return out[0, 0]


# speedup vs baseline: 1.0626x; 1.0626x over previous
import jax
import jax.numpy as jnp
from jax.experimental import pallas as pl
from jax.experimental.pallas import tpu as pltpu

_K = 10
_GAMMA = 8.0
_N = 10000
_NP = 10240
_D = 128
_T = 256
_TA = 16
_TQ = 2 * _TA
_CW = 512
_NC = _NP // _CW
_NR = _D // 8


def _body(a1_ref, a2_ref, g1p_ref, g2p_ref, loss_ref,
          q1_ref, q2_ref, bq_ref, dist_ref, accum_ref, sg_ref,
          g1t_ref, g2t_ref):
    t = pl.program_id(0)

    @pl.when(t == 0)
    def _init():
        accum_ref[...] = jnp.zeros_like(accum_ref)
        g1t_ref[:, 0:_N] = g1p_ref[...].T
        g2t_ref[:, 0:_N] = g2p_ref[...].T
        padfill = jnp.full((_D, _NP - _N), jnp.inf, dtype=jnp.float32)
        g1t_ref[:, _N:] = padfill
        g2t_ref[:, _N:] = padfill
        sg_ref[0:1, :] = jnp.sum(g2t_ref[...], axis=0, keepdims=True)
        sg_ref[1:2, :] = jnp.sum(g1t_ref[...], axis=0, keepdims=True)

    for i in range(_TA):
        i1 = a1_ref[t * _TA + i]
        i2 = a2_ref[t * _TA + i]
        q1_ref[pl.ds(i, 1), :] = g1p_ref[pl.ds(i1, 1), :]
        q2_ref[pl.ds(i, 1), :] = g2p_ref[pl.ds(i2, 1), :]

    q1 = q1_ref[...]
    q2 = q2_ref[...]
    dvec = jnp.sum(jnp.abs(q1 - q2), axis=1, keepdims=True) + _GAMMA
    q1t = q1.T
    q2t = q2.T
    for j in range(_TA):
        bq_ref[j] = jnp.broadcast_to(q1t[:, j:j + 1], (_D, _CW))
        bq_ref[_TA + j] = jnp.broadcast_to(q2t[:, j:j + 1], (_D, _CW))

    sq1 = jnp.sum(q1, axis=1, keepdims=True)
    sq2 = jnp.sum(q2, axis=1, keepdims=True)
    for d, (gref, sq) in enumerate(((g2t_ref, sq1), (g1t_ref, sq2))):
        for g0 in range(0, _TA, 8):
            def chunk(c, _, gref=gref, sq=sq, d=d, g0=g0):
                lanes = pl.ds(c * _CW, _CW)
                accs = [None] * 8
                for r in range(_NR):
                    blk = gref[pl.ds(8 * r, 8), lanes]
                    for j in range(8):
                        bq = bq_ref[d * _TA + g0 + j, pl.ds(8 * r, 8), :]
                        term = jnp.minimum(blk, bq)
                        accs[j] = term if accs[j] is None else accs[j] + term
                sg = sg_ref[pl.ds(d, 1), lanes]
                for j in range(8):
                    red = jnp.sum(accs[j], axis=0, keepdims=True)
                    dist_ref[pl.ds(d * _TA + g0 + j, 1), lanes] = (
                        sq[g0 + j:g0 + j + 1, :] + sg - 2.0 * red)
                return 0
            jax.lax.fori_loop(0, _NC, chunk, 0, unroll=_NC)

    dist = dist_ref[...]
    dmat = jnp.concatenate([dvec, dvec], axis=0)
    rem = jnp.full((_TQ, 1), float(_K), dtype=jnp.float32)
    av = jnp.zeros((_TQ, 1), dtype=jnp.float32)
    for _ in range(_K):
        m = jnp.min(dist, axis=1, keepdims=True)
        eq = dist == m
        cnt = jnp.sum(eq.astype(jnp.float32), axis=1, keepdims=True)
        dist = jnp.where(eq, jnp.inf, dist)
        w = jnp.minimum(cnt, rem)
        rem = rem - w
        av = av + jnp.maximum(dmat - m, 0.0) * w
    accum_ref[...] += av

    @pl.when(t == pl.num_programs(0) - 1)
    def _fin():
        loss_ref[0, 0] = jnp.sum(accum_ref[...]) / (_T * _K)


def kernel(out1, out2, anchor1, anchor2):
    out = pl.pallas_call(
        _body,
        grid=(_T // _TA,),
        in_specs=[
            pl.BlockSpec(memory_space=pltpu.SMEM),
            pl.BlockSpec(memory_space=pltpu.SMEM),
            pl.BlockSpec((_N, _D), lambda t: (0, 0)),
            pl.BlockSpec((_N, _D), lambda t: (0, 0)),
        ],
        out_specs=pl.BlockSpec(memory_space=pltpu.SMEM),
        out_shape=jax.ShapeDtypeStruct((1, 1), jnp.float32),
        scratch_shapes=[
            pltpu.VMEM((_TA, _D), jnp.float32),
            pltpu.VMEM((_TA, _D), jnp.float32),
            pltpu.VMEM((_TQ, _D, _CW), jnp.float32),
            pltpu.VMEM((_TQ, _NP), jnp.float32),
            pltpu.VMEM((_TQ, 1), jnp.float32),
            pltpu.VMEM((2, _NP), jnp.float32),
            pltpu.VMEM((_D, _NP), jnp.float32),
            pltpu.VMEM((_D, _NP), jnp.float32),
        ],
    )(anchor1, anchor2, out1, out2)
    return out[0, 0]
